# baseline (speedup 1.0000x reference)
import functools
import math

import jax
import jax.numpy as jnp
from jax import lax
from jax.experimental import pallas as pl
from jax.experimental.pallas import tpu as pltpu
from jax.experimental.pallas import tpu_sc as plsc

DT = 0.01
COUPLING_STRENGTH = 2.0
TWO_PI = 2.0 * math.pi
INV_TWO_PI = 1.0 / TWO_PI

NUM_CORES = 2
NUM_SUBCORES = 16
LANES = 16

B, N, K = 64, 10000, 16
BPW = B // NUM_SUBCORES
HALF = N // NUM_CORES
W = 1024
SLAB_STARTS = (0, 1024, 2048, 3072, HALF - W)


def _tc_elemwise_body(mu_ref, phase_ref, amp_ref, freq_ref,
                      s_ref, c_ref, base_ref, amp_out_ref):
    p = phase_ref[...]
    s_ref[...] = jnp.sin(p)
    c_ref[...] = jnp.cos(p)
    f = freq_ref[...]
    base_ref[...] = p + (TWO_PI * f) * DT
    a = amp_ref[...]
    mu = mu_ref[0]
    na = a + DT * a * (mu - a * a)
    amp_out_ref[...] = jnp.clip(na, 1e-06, 10.0)


def _tc_elemwise(phase, amplitude, frequencies, mu):
    out_shape = [
        jax.ShapeDtypeStruct((B, N), jnp.float32),
        jax.ShapeDtypeStruct((B, N), jnp.float32),
        jax.ShapeDtypeStruct((B, N), jnp.float32),
        jax.ShapeDtypeStruct((B, N), jnp.float32),
    ]
    return pl.pallas_call(
        _tc_elemwise_body,
        out_shape=out_shape,
        in_specs=[
            pl.BlockSpec(memory_space=pltpu.SMEM),
            pl.BlockSpec(memory_space=pltpu.VMEM),
            pl.BlockSpec(memory_space=pltpu.VMEM),
            pl.BlockSpec(memory_space=pltpu.VMEM),
        ],
        out_specs=[pl.BlockSpec(memory_space=pltpu.VMEM)] * 4,
    )(jnp.reshape(mu.astype(jnp.float32), (1,)), phase, amplitude,
      jnp.reshape(frequencies, (1, N)))


def _sc_body(s_hbm, c_hbm, base_hbm, nbr_hbm, out_hbm,
             s_tab, c_tab, idx_b, base_b, out_b,
             tab_sem, in_sem0, in_sem1, out_sem0, out_sem1):
    in_sems = (in_sem0, in_sem1)
    out_sems = (out_sem0, out_sem1)
    cid = lax.axis_index("c")
    sid = lax.axis_index("s")
    b0 = sid * BPW

    def slab_in(si, buf):
        start = cid * HALF + SLAB_STARTS[si]
        cps = [pltpu.async_copy(
            nbr_hbm.at[pl.ds(start * K, W * K)], idx_b.at[buf], in_sems[buf])]
        for i in range(BPW):
            cps.append(pltpu.async_copy(
                base_hbm.at[b0 + i, pl.ds(start, W)], base_b.at[buf, i],
                in_sems[buf]))
        return cps

    tab_cps = []
    for i in range(BPW):
        tab_cps.append(pltpu.async_copy(s_hbm.at[b0 + i], s_tab.at[i], tab_sem))
        tab_cps.append(pltpu.async_copy(c_hbm.at[b0 + i], c_tab.at[i], tab_sem))
    in_cps = slab_in(0, 0)
    for cp in tab_cps:
        cp.wait()

    out_cps = [None, None]
    n_slabs = len(SLAB_STARTS)
    for si in range(n_slabs):
        buf = si % 2
        start = cid * HALF + SLAB_STARTS[si]
        for cp in in_cps:
            cp.wait()
        if si + 1 < n_slabs:
            in_cps = slab_in(si + 1, 1 - buf)
        if out_cps[buf] is not None:
            for cp in out_cps[buf]:
                cp.wait()

        @plsc.parallel_loop(0, W // LANES)
        def block_body(blk):
            nl = blk * LANES
            gcol = start + nl
            koff = K * nl
            ivs = []
            for k in range(K):
                kvec = lax.iota(jnp.int32, LANES) * K + (k + koff)
                ivs.append(plsc.load_gather(idx_b.at[buf], [kvec]))
            for b in range(BPW):
                bv = jnp.full((LANES,), b, jnp.int32)
                acc_s = plsc.load_gather(s_tab, [bv, ivs[0]])
                acc_c = plsc.load_gather(c_tab, [bv, ivs[0]])
                for k in range(1, K):
                    acc_s = acc_s + plsc.load_gather(s_tab, [bv, ivs[k]])
                    acc_c = acc_c + plsc.load_gather(c_tab, [bv, ivs[k]])
                sv = s_tab[b, pl.ds(gcol, LANES)]
                cv = c_tab[b, pl.ds(gcol, LANES)]
                coup = (COUPLING_STRENGTH / K) * (cv * acc_s - sv * acc_c)
                t = base_b[buf, b, pl.ds(nl, LANES)] + DT * coup
                q0 = t * INV_TWO_PI
                qf = q0.astype(jnp.int32).astype(jnp.float32)
                q = jnp.where(qf > q0, qf - 1.0, qf)
                out_b[buf, b, pl.ds(nl, LANES)] = t - q * TWO_PI

        cps = []
        for i in range(BPW):
            cps.append(pltpu.async_copy(
                out_b.at[buf, i], out_hbm.at[b0 + i, pl.ds(start, W)],
                out_sems[buf]))
        out_cps[buf] = cps

    for cps in out_cps:
        if cps is not None:
            for cp in cps:
                cp.wait()


def _sc_gather(s, c, base, nbr_flat):
    mesh = plsc.VectorSubcoreMesh(
        core_axis_name="c", subcore_axis_name="s",
        num_cores=NUM_CORES, num_subcores=NUM_SUBCORES)
    return pl.kernel(
        _sc_body,
        out_type=jax.ShapeDtypeStruct((B, N), jnp.float32),
        mesh=mesh,
        compiler_params=pltpu.CompilerParams(
            use_tc_tiling_on_sc=False, needs_layout_passes=False),
        scratch_types=[
            pltpu.VMEM((BPW, N), jnp.float32),
            pltpu.VMEM((BPW, N), jnp.float32),
            pltpu.VMEM((2, W * K), jnp.int32),
            pltpu.VMEM((2, BPW, W), jnp.float32),
            pltpu.VMEM((2, BPW, W), jnp.float32),
            pltpu.SemaphoreType.DMA,
            pltpu.SemaphoreType.DMA,
            pltpu.SemaphoreType.DMA,
            pltpu.SemaphoreType.DMA,
            pltpu.SemaphoreType.DMA,
        ],
    )(s, c, base, nbr_flat)


def kernel(phase, amplitude, frequencies, mu, neighbors):
    nbr_flat = jnp.reshape(neighbors, (N * K,))
    s, c, base, new_amp = _tc_elemwise(phase, amplitude, frequencies, mu)
    new_phase = _sc_gather(s, c, base, nbr_flat)
    return new_phase, new_amp


# speedup vs baseline: 1.2908x; 1.2908x over previous
import functools
import math

import jax
import jax.numpy as jnp
from jax import lax
from jax.experimental import pallas as pl
from jax.experimental.pallas import tpu as pltpu
from jax.experimental.pallas import tpu_sc as plsc

DT = 0.01
COUPLING_STRENGTH = 2.0
TWO_PI = 2.0 * math.pi
INV_TWO_PI = 1.0 / TWO_PI
FIX = 512.0
INV_FIX = 1.0 / FIX

NUM_CORES = 2
NUM_SUBCORES = 16
LANES = 16

B, N, K = 64, 10000, 16
BPW = B // NUM_SUBCORES
HALF = N // NUM_CORES
W = 1280
SLAB_STARTS = (0, 1280, 2560, HALF - W)


def _tc_elemwise_body(mu_ref, phase_ref, amp_ref, freq_ref,
                      q_ref, base_ref, amp_out_ref):
    p = phase_ref[...]
    sq = ((jnp.sin(p) + 1.0) * FIX + 0.5).astype(jnp.int32)
    cq = ((jnp.cos(p) + 1.0) * FIX + 0.5).astype(jnp.int32)
    q_ref[...] = (cq << 16) | sq
    f = freq_ref[...]
    base_ref[...] = p + (TWO_PI * f) * DT
    a = amp_ref[...]
    mu = mu_ref[0]
    na = a + DT * a * (mu - a * a)
    amp_out_ref[...] = jnp.clip(na, 1e-06, 10.0)


def _tc_elemwise(phase, amplitude, frequencies, mu):
    out_shape = [
        jax.ShapeDtypeStruct((B, N), jnp.int32),
        jax.ShapeDtypeStruct((B, N), jnp.float32),
        jax.ShapeDtypeStruct((B, N), jnp.float32),
    ]
    return pl.pallas_call(
        _tc_elemwise_body,
        out_shape=out_shape,
        in_specs=[
            pl.BlockSpec(memory_space=pltpu.SMEM),
            pl.BlockSpec(memory_space=pltpu.VMEM),
            pl.BlockSpec(memory_space=pltpu.VMEM),
            pl.BlockSpec(memory_space=pltpu.VMEM),
        ],
        out_specs=[pl.BlockSpec(memory_space=pltpu.VMEM)] * 3,
    )(jnp.reshape(mu.astype(jnp.float32), (1,)), phase, amplitude,
      jnp.reshape(frequencies, (1, N)))


def _unbias(word, scale, bias):
    lo = (word & 0xFFFF).astype(jnp.float32) * scale - bias
    hi = (word >> 16).astype(jnp.float32) * scale - bias
    return lo, hi


def _sc_body(q_hbm, base_hbm, nbr_hbm, out_hbm,
             q_tab, idx_b, base_b, out_b,
             tab_sem, in_sem0, in_sem1, out_sem0, out_sem1):
    in_sems = (in_sem0, in_sem1)
    out_sems = (out_sem0, out_sem1)
    cid = lax.axis_index("c")
    sid = lax.axis_index("s")
    b0 = sid * BPW

    def slab_in(si, buf):
        start = cid * HALF + SLAB_STARTS[si]
        cps = [pltpu.async_copy(
            nbr_hbm.at[pl.ds(start * K, W * K)], idx_b.at[buf], in_sems[buf])]
        for i in range(BPW):
            cps.append(pltpu.async_copy(
                base_hbm.at[b0 + i, pl.ds(start, W)], base_b.at[buf, i],
                in_sems[buf]))
        return cps

    tab_cps = [pltpu.async_copy(q_hbm.at[b0 + i], q_tab.at[i], tab_sem)
               for i in range(BPW)]
    in_cps = slab_in(0, 0)
    for cp in tab_cps:
        cp.wait()

    out_cps = [None, None]
    n_slabs = len(SLAB_STARTS)
    for si in range(n_slabs):
        buf = si % 2
        start = cid * HALF + SLAB_STARTS[si]
        for cp in in_cps:
            cp.wait()
        if si + 1 < n_slabs:
            in_cps = slab_in(si + 1, 1 - buf)
        if out_cps[buf] is not None:
            for cp in out_cps[buf]:
                cp.wait()

        @plsc.parallel_loop(0, W // LANES)
        def block_body(blk):
            nl = blk * LANES
            gcol = start + nl
            koff = K * nl
            ivs = []
            for k in range(K):
                kvec = lax.iota(jnp.int32, LANES) * K + (k + koff)
                ivs.append(plsc.load_gather(idx_b.at[buf], [kvec]))
            for b in range(BPW):
                bv = jnp.full((LANES,), b, jnp.int32)
                acc = plsc.load_gather(q_tab, [bv, ivs[0]])
                for k in range(1, K):
                    acc = acc + plsc.load_gather(q_tab, [bv, ivs[k]])
                s_sum, c_sum = _unbias(acc, INV_FIX, float(K))
                sv, cv = _unbias(q_tab[b, pl.ds(gcol, LANES)], INV_FIX, 1.0)
                coup = cv * s_sum - sv * c_sum
                t = base_b[buf, b, pl.ds(nl, LANES)] + \
                    (DT * COUPLING_STRENGTH / K) * coup
                q0 = t * INV_TWO_PI
                qf = q0.astype(jnp.int32).astype(jnp.float32)
                qfl = jnp.where(qf > q0, qf - 1.0, qf)
                out_b[buf, b, pl.ds(nl, LANES)] = t - qfl * TWO_PI

        cps = []
        for i in range(BPW):
            cps.append(pltpu.async_copy(
                out_b.at[buf, i], out_hbm.at[b0 + i, pl.ds(start, W)],
                out_sems[buf]))
        out_cps[buf] = cps

    for cps in out_cps:
        if cps is not None:
            for cp in cps:
                cp.wait()


def _sc_gather(q, base, nbr_flat):
    mesh = plsc.VectorSubcoreMesh(
        core_axis_name="c", subcore_axis_name="s",
        num_cores=NUM_CORES, num_subcores=NUM_SUBCORES)
    return pl.kernel(
        _sc_body,
        out_type=jax.ShapeDtypeStruct((B, N), jnp.float32),
        mesh=mesh,
        compiler_params=pltpu.CompilerParams(
            use_tc_tiling_on_sc=False, needs_layout_passes=False),
        scratch_types=[
            pltpu.VMEM((BPW, N), jnp.int32),
            pltpu.VMEM((2, W * K), jnp.int32),
            pltpu.VMEM((2, BPW, W), jnp.float32),
            pltpu.VMEM((2, BPW, W), jnp.float32),
            pltpu.SemaphoreType.DMA,
            pltpu.SemaphoreType.DMA,
            pltpu.SemaphoreType.DMA,
            pltpu.SemaphoreType.DMA,
            pltpu.SemaphoreType.DMA,
        ],
    )(q, base, nbr_flat)


def kernel(phase, amplitude, frequencies, mu, neighbors):
    nbr_flat = jnp.reshape(neighbors, (N * K,))
    q, base, new_amp = _tc_elemwise(phase, amplitude, frequencies, mu)
    new_phase = _sc_gather(q, base, nbr_flat)
    return new_phase, new_amp


# speedup vs baseline: 1.3210x; 1.0234x over previous
import functools
import math

import jax
import jax.numpy as jnp
from jax import lax
from jax.experimental import pallas as pl
from jax.experimental.pallas import tpu as pltpu
from jax.experimental.pallas import tpu_sc as plsc

DT = 0.01
COUPLING_STRENGTH = 2.0
TWO_PI = 2.0 * math.pi
INV_TWO_PI = 1.0 / TWO_PI
FIX = 512.0
INV_FIX = 1.0 / FIX

NUM_CORES = 2
NUM_SUBCORES = 16
LANES = 16

B, N, K = 64, 10000, 16
BPW = B // NUM_SUBCORES
HALF = N // NUM_CORES
W = 1280
SLAB_STARTS = (0, 1280, 2560, HALF - W)


def _tc_elemwise_body(mu_ref, phase_ref, amp_ref, freq_ref,
                      q_ref, base_ref, amp_out_ref):
    p = phase_ref[...]
    sq = ((jnp.sin(p) + 1.0) * FIX + 0.5).astype(jnp.int32)
    cq = ((jnp.cos(p) + 1.0) * FIX + 0.5).astype(jnp.int32)
    q_ref[...] = (cq << 16) | sq
    f = freq_ref[...]
    base_ref[...] = p + (TWO_PI * f) * DT
    a = amp_ref[...]
    mu = mu_ref[0]
    na = a + DT * a * (mu - a * a)
    amp_out_ref[...] = jnp.clip(na, 1e-06, 10.0)


TC_BLK = 1280


def _tc_elemwise(phase, amplitude, frequencies, mu):
    out_shape = [
        jax.ShapeDtypeStruct((B, N), jnp.int32),
        jax.ShapeDtypeStruct((B, N), jnp.float32),
        jax.ShapeDtypeStruct((B, N), jnp.float32),
    ]
    bspec = pl.BlockSpec((B, TC_BLK), lambda i: (0, i))
    return pl.pallas_call(
        _tc_elemwise_body,
        grid=(pl.cdiv(N, TC_BLK),),
        out_shape=out_shape,
        in_specs=[
            pl.BlockSpec(memory_space=pltpu.SMEM),
            bspec,
            bspec,
            pl.BlockSpec((1, TC_BLK), lambda i: (0, i)),
        ],
        out_specs=[bspec] * 3,
    )(jnp.reshape(mu.astype(jnp.float32), (1,)), phase, amplitude,
      jnp.reshape(frequencies, (1, N)))


def _unbias(word, scale, bias):
    lo = (word & 0xFFFF).astype(jnp.float32) * scale - bias
    hi = (word >> 16).astype(jnp.float32) * scale - bias
    return lo, hi


def _sc_body(q_hbm, base_hbm, nbr_hbm, out_hbm,
             q_tab, idx_b, base_b, out_b,
             tab_sem, in_sem0, in_sem1, out_sem0, out_sem1):
    in_sems = (in_sem0, in_sem1)
    out_sems = (out_sem0, out_sem1)
    cid = lax.axis_index("c")
    sid = lax.axis_index("s")
    b0 = sid * BPW

    def slab_in(si, buf):
        start = cid * HALF + SLAB_STARTS[si]
        cps = [pltpu.async_copy(
            nbr_hbm.at[pl.ds(start * K, W * K)], idx_b.at[buf], in_sems[buf])]
        for i in range(BPW):
            cps.append(pltpu.async_copy(
                base_hbm.at[b0 + i, pl.ds(start, W)], base_b.at[buf, i],
                in_sems[buf]))
        return cps

    tab_cps = [pltpu.async_copy(q_hbm.at[b0 + i], q_tab.at[i], tab_sem)
               for i in range(BPW)]
    in_cps = slab_in(0, 0)
    for cp in tab_cps:
        cp.wait()

    out_cps = [None, None]
    n_slabs = len(SLAB_STARTS)
    for si in range(n_slabs):
        buf = si % 2
        start = cid * HALF + SLAB_STARTS[si]
        for cp in in_cps:
            cp.wait()
        if si + 1 < n_slabs:
            in_cps = slab_in(si + 1, 1 - buf)
        if out_cps[buf] is not None:
            for cp in out_cps[buf]:
                cp.wait()

        @plsc.parallel_loop(0, W // LANES)
        def block_body(blk):
            nl = blk * LANES
            gcol = start + nl
            koff = K * nl
            ivs = []
            for k in range(K):
                kvec = lax.iota(jnp.int32, LANES) * K + (k + koff)
                ivs.append(plsc.load_gather(idx_b.at[buf], [kvec]))
            for b in range(BPW):
                bv = jnp.full((LANES,), b, jnp.int32)
                acc = plsc.load_gather(q_tab, [bv, ivs[0]])
                for k in range(1, K):
                    acc = acc + plsc.load_gather(q_tab, [bv, ivs[k]])
                s_sum, c_sum = _unbias(acc, INV_FIX, float(K))
                sv, cv = _unbias(q_tab[b, pl.ds(gcol, LANES)], INV_FIX, 1.0)
                coup = cv * s_sum - sv * c_sum
                t = base_b[buf, b, pl.ds(nl, LANES)] + \
                    (DT * COUPLING_STRENGTH / K) * coup
                q0 = t * INV_TWO_PI
                qf = q0.astype(jnp.int32).astype(jnp.float32)
                qfl = jnp.where(qf > q0, qf - 1.0, qf)
                out_b[buf, b, pl.ds(nl, LANES)] = t - qfl * TWO_PI

        cps = []
        for i in range(BPW):
            cps.append(pltpu.async_copy(
                out_b.at[buf, i], out_hbm.at[b0 + i, pl.ds(start, W)],
                out_sems[buf]))
        out_cps[buf] = cps

    for cps in out_cps:
        if cps is not None:
            for cp in cps:
                cp.wait()


def _sc_gather(q, base, nbr_flat):
    mesh = plsc.VectorSubcoreMesh(
        core_axis_name="c", subcore_axis_name="s",
        num_cores=NUM_CORES, num_subcores=NUM_SUBCORES)
    return pl.kernel(
        _sc_body,
        out_type=jax.ShapeDtypeStruct((B, N), jnp.float32),
        mesh=mesh,
        compiler_params=pltpu.CompilerParams(
            use_tc_tiling_on_sc=False, needs_layout_passes=False),
        scratch_types=[
            pltpu.VMEM((BPW, N), jnp.int32),
            pltpu.VMEM((2, W * K), jnp.int32),
            pltpu.VMEM((2, BPW, W), jnp.float32),
            pltpu.VMEM((2, BPW, W), jnp.float32),
            pltpu.SemaphoreType.DMA,
            pltpu.SemaphoreType.DMA,
            pltpu.SemaphoreType.DMA,
            pltpu.SemaphoreType.DMA,
            pltpu.SemaphoreType.DMA,
        ],
    )(q, base, nbr_flat)


def kernel(phase, amplitude, frequencies, mu, neighbors):
    nbr_flat = jnp.reshape(neighbors, (N * K,))
    q, base, new_amp = _tc_elemwise(phase, amplitude, frequencies, mu)
    new_phase = _sc_gather(q, base, nbr_flat)
    return new_phase, new_amp


# speedup vs baseline: 1.4403x; 1.0903x over previous
import functools
import math

import jax
import jax.numpy as jnp
from jax import lax
from jax.experimental import pallas as pl
from jax.experimental.pallas import tpu as pltpu
from jax.experimental.pallas import tpu_sc as plsc

DT = 0.01
COUPLING_STRENGTH = 2.0
TWO_PI = 2.0 * math.pi
FIX = 512.0
INV_FIX = 1.0 / FIX

NUM_CORES = 2
NUM_SUBCORES = 16
LANES = 16

B, N, K = 64, 10000, 16
BPW = 8
NCHUNK = B // BPW
W = 1264
N_SLABS = 8
LAST_START = N - W


def _tc_pack_body(mu_ref, phase_ref, amp_ref, q_ref, amp_out_ref):
    p = phase_ref[...]
    sq = ((jnp.sin(p) + 1.0) * FIX + 0.5).astype(jnp.int32)
    cq = ((jnp.cos(p) + 1.0) * FIX + 0.5).astype(jnp.int32)
    q_ref[...] = (cq << 16) | sq
    a = amp_ref[...]
    mu = mu_ref[0]
    na = a + DT * a * (mu - a * a)
    amp_out_ref[...] = jnp.clip(na, 1e-06, 10.0)


TC_BLK = 1280


def _tc_pack(phase, amplitude, mu):
    out_shape = [
        jax.ShapeDtypeStruct((B, N), jnp.int32),
        jax.ShapeDtypeStruct((B, N), jnp.float32),
    ]
    bspec = pl.BlockSpec((B, TC_BLK), lambda i: (0, i))
    return pl.pallas_call(
        _tc_pack_body,
        grid=(pl.cdiv(N, TC_BLK),),
        out_shape=out_shape,
        in_specs=[
            pl.BlockSpec(memory_space=pltpu.SMEM),
            bspec,
            bspec,
        ],
        out_specs=[bspec] * 2,
    )(jnp.reshape(mu.astype(jnp.float32), (1,)), phase, amplitude)


def _sc_body(q_hbm, nbr_hbm, out_hbm,
             q_tab, idx_b0,
             ob0, ob1, ob2, ob3, ob4, ob5, ob6, ob7,
             tab_sem, in_sem0, in_sem1, out_sem):
    out_bufs = (ob0, ob1, ob2, ob3, ob4, ob5, ob6, ob7)
    idx_bufs = (idx_b0, idx_b0)
    in_sems = (in_sem0, in_sem1)
    cid = lax.axis_index("c")
    sid = lax.axis_index("s")
    wid = sid * NUM_CORES + cid
    chunk = wid % NCHUNK
    sw = wid // NCHUNK
    b0 = chunk * BPW

    def slab_start(slab_id):
        return jnp.where(slab_id == N_SLABS - 1, LAST_START, slab_id * W)

    def slab_in(slab_id, buf):
        start = slab_start(slab_id)
        return [pltpu.async_copy(
            nbr_hbm.at[pl.ds(start * K, W * K)], idx_bufs[buf],
            in_sems[buf])]

    tab_cp = pltpu.async_copy(q_hbm.at[pl.ds(b0, BPW)], q_tab, tab_sem)
    in_cps = slab_in(sw, 0)
    tab_cp.wait()

    out_cp = None
    for si in range(2):
        buf = si % 2
        slab_id = sw + 4 * si
        start = slab_start(slab_id)
        for cp in in_cps:
            cp.wait()
        if out_cp is not None:
            for cp in out_cp:
                cp.wait()

        @plsc.parallel_loop(0, W // LANES)
        def block_body(blk):
            nl = blk * LANES
            gcol = start + nl
            koff = K * nl
            ivs = []
            for k in range(K):
                kvec = lax.iota(jnp.int32, LANES) * K + (k + koff)
                ivs.append(plsc.load_gather(idx_bufs[buf], [kvec]))
            for b in range(BPW):
                bv = jnp.full((LANES,), b, jnp.int32)
                acc = plsc.load_gather(q_tab, [bv, ivs[0]])
                for k in range(1, K):
                    acc = acc + plsc.load_gather(q_tab, [bv, ivs[k]])
                s_sum = (acc & 0xFFFF).astype(jnp.float32) * INV_FIX - float(K)
                c_sum = (acc >> 16).astype(jnp.float32) * INV_FIX - float(K)
                w = q_tab[b, pl.ds(gcol, LANES)]
                sv = (w & 0xFFFF).astype(jnp.float32) * INV_FIX - 1.0
                cv = (w >> 16).astype(jnp.float32) * INV_FIX - 1.0
                coup = (COUPLING_STRENGTH / K) * (cv * s_sum - sv * c_sum)
                out_bufs[b][pl.ds(nl, LANES)] = coup

        if si == 0:
            in_cps = slab_in(sw + 4, 1)
        cps = []
        for i in range(BPW):
            cps.append(pltpu.async_copy(
                out_bufs[i], out_hbm.at[pl.ds((b0 + i) * N + start, W)],
                out_sem))
        out_cp = cps

    for cp in out_cp:
        cp.wait()


def _sc_coupling(q, nbr_flat):
    mesh = plsc.VectorSubcoreMesh(
        core_axis_name="c", subcore_axis_name="s",
        num_cores=NUM_CORES, num_subcores=NUM_SUBCORES)
    return pl.kernel(
        _sc_body,
        out_type=jax.ShapeDtypeStruct((B * N,), jnp.float32),
        mesh=mesh,
        compiler_params=pltpu.CompilerParams(needs_layout_passes=False),
        scratch_types=[
            pltpu.VMEM((BPW, N), jnp.int32),
            pltpu.VMEM((W * K,), jnp.int32),
        ] + [pltpu.VMEM((W,), jnp.float32)] * BPW + [
            pltpu.SemaphoreType.DMA,
            pltpu.SemaphoreType.DMA,
            pltpu.SemaphoreType.DMA,
            pltpu.SemaphoreType.DMA,
        ],
    )(q, nbr_flat)


def kernel(phase, amplitude, frequencies, mu, neighbors):
    nbr_flat = jnp.reshape(neighbors, (N * K,))
    q, new_amp = _tc_pack(phase, amplitude, mu)
    coup = jnp.reshape(_sc_coupling(q, nbr_flat), (B, N))
    new_phase = jnp.mod(phase + TWO_PI * frequencies * DT + DT * coup, TWO_PI)
    return new_phase, new_amp


# speedup vs baseline: 1.5142x; 1.0513x over previous
import functools
import math

import jax
import jax.numpy as jnp
from jax import lax
from jax.experimental import pallas as pl
from jax.experimental.pallas import tpu as pltpu
from jax.experimental.pallas import tpu_sc as plsc

DT = 0.01
COUPLING_STRENGTH = 2.0
TWO_PI = 2.0 * math.pi
FIX = 512.0
INV_FIX = 1.0 / FIX

NUM_CORES = 2
NUM_SUBCORES = 16
LANES = 16

B, N, K = 64, 10000, 16
BPW = 8
NCHUNK = B // BPW
W = 1264
N_SLABS = 8
LAST_START = N - W


def _tc_pack_body(mu_ref, phase_ref, amp_ref, q_ref, amp_out_ref):
    p = phase_ref[...]
    qd = (p * (2.0 / math.pi)).astype(jnp.int32)
    r = p - qd.astype(jnp.float32) * (math.pi / 2.0)
    r2 = r * r
    s0 = r * (1.0 + r2 * (-1.0 / 6.0 + r2 * (1.0 / 120.0
                                             + r2 * (-1.0 / 5040.0))))
    c0 = 1.0 + r2 * (-0.5 + r2 * (1.0 / 24.0 + r2 * (-1.0 / 720.0
                                                     + r2 * (1.0 / 40320.0))))
    flip = qd >= 2
    swap = (qd == 1) | (qd == 3)
    s1 = jnp.where(swap, c0, s0)
    c1 = jnp.where(swap, -s0, c0)
    sgn = jnp.where(flip, -1.0, 1.0)
    sq = (sgn * s1 * FIX + (FIX + 0.5)).astype(jnp.int32)
    cq = (sgn * c1 * FIX + (FIX + 0.5)).astype(jnp.int32)
    q_ref[...] = (cq << 16) | sq
    a = amp_ref[...]
    mu = mu_ref[0]
    na = a + DT * a * (mu - a * a)
    amp_out_ref[...] = jnp.clip(na, 1e-06, 10.0)


TC_BLK = 1280


def _tc_pack(phase, amplitude, mu):
    out_shape = [
        jax.ShapeDtypeStruct((B, N), jnp.int32),
        jax.ShapeDtypeStruct((B, N), jnp.float32),
    ]
    bspec = pl.BlockSpec((B, TC_BLK), lambda i: (0, i))
    return pl.pallas_call(
        _tc_pack_body,
        grid=(pl.cdiv(N, TC_BLK),),
        out_shape=out_shape,
        in_specs=[
            pl.BlockSpec(memory_space=pltpu.SMEM),
            bspec,
            bspec,
        ],
        out_specs=[bspec] * 2,
    )(jnp.reshape(mu.astype(jnp.float32), (1,)), phase, amplitude)


def _sc_body(q_hbm, nbr_hbm, out_hbm,
             q_tab, idx_b0,
             ob0, ob1, ob2, ob3, ob4, ob5, ob6, ob7,
             tab_sem, in_sem0, in_sem1, out_sem):
    out_bufs = (ob0, ob1, ob2, ob3, ob4, ob5, ob6, ob7)
    idx_bufs = (idx_b0, idx_b0)
    in_sems = (in_sem0, in_sem1)
    cid = lax.axis_index("c")
    sid = lax.axis_index("s")
    wid = sid * NUM_CORES + cid
    chunk = wid % NCHUNK
    sw = wid // NCHUNK
    b0 = chunk * BPW

    def slab_start(slab_id):
        return jnp.where(slab_id == N_SLABS - 1, LAST_START, slab_id * W)

    def slab_in(slab_id, buf):
        start = slab_start(slab_id)
        return [pltpu.async_copy(
            nbr_hbm.at[pl.ds(start * K, W * K)], idx_bufs[buf],
            in_sems[buf])]

    tab_cp = pltpu.async_copy(q_hbm.at[pl.ds(b0, BPW)], q_tab, tab_sem)
    in_cps = slab_in(sw, 0)
    tab_cp.wait()

    out_cp = None
    for si in range(2):
        buf = si % 2
        slab_id = sw + 4 * si
        start = slab_start(slab_id)
        for cp in in_cps:
            cp.wait()
        if out_cp is not None:
            for cp in out_cp:
                cp.wait()

        @plsc.parallel_loop(0, W // LANES)
        def block_body(blk):
            nl = blk * LANES
            gcol = start + nl
            koff = K * nl
            ivs = []
            for k in range(K):
                kvec = lax.iota(jnp.int32, LANES) * K + (k + koff)
                ivs.append(plsc.load_gather(idx_bufs[buf], [kvec]))
            for b in range(BPW):
                bv = jnp.full((LANES,), b, jnp.int32)
                acc = plsc.load_gather(q_tab, [bv, ivs[0]])
                for k in range(1, K):
                    acc = acc + plsc.load_gather(q_tab, [bv, ivs[k]])
                s_sum = (acc & 0xFFFF).astype(jnp.float32) * INV_FIX - float(K)
                c_sum = (acc >> 16).astype(jnp.float32) * INV_FIX - float(K)
                w = q_tab[b, pl.ds(gcol, LANES)]
                sv = (w & 0xFFFF).astype(jnp.float32) * INV_FIX - 1.0
                cv = (w >> 16).astype(jnp.float32) * INV_FIX - 1.0
                coup = (COUPLING_STRENGTH / K) * (cv * s_sum - sv * c_sum)
                out_bufs[b][pl.ds(nl, LANES)] = coup

        if si == 0:
            in_cps = slab_in(sw + 4, 1)
        cps = []
        for i in range(BPW):
            cps.append(pltpu.async_copy(
                out_bufs[i], out_hbm.at[pl.ds((b0 + i) * N + start, W)],
                out_sem))
        out_cp = cps

    for cp in out_cp:
        cp.wait()


def _sc_coupling(q, nbr_flat):
    mesh = plsc.VectorSubcoreMesh(
        core_axis_name="c", subcore_axis_name="s",
        num_cores=NUM_CORES, num_subcores=NUM_SUBCORES)
    return pl.kernel(
        _sc_body,
        out_type=jax.ShapeDtypeStruct((B * N,), jnp.float32),
        mesh=mesh,
        compiler_params=pltpu.CompilerParams(needs_layout_passes=False),
        scratch_types=[
            pltpu.VMEM((BPW, N), jnp.int32),
            pltpu.VMEM((W * K,), jnp.int32),
        ] + [pltpu.VMEM((W,), jnp.float32)] * BPW + [
            pltpu.SemaphoreType.DMA,
            pltpu.SemaphoreType.DMA,
            pltpu.SemaphoreType.DMA,
            pltpu.SemaphoreType.DMA,
        ],
    )(q, nbr_flat)


def kernel(phase, amplitude, frequencies, mu, neighbors):
    nbr_flat = jnp.reshape(neighbors, (N * K,))
    q, new_amp = _tc_pack(phase, amplitude, mu)
    coup = jnp.reshape(_sc_coupling(q, nbr_flat), (B, N))
    new_phase = jnp.mod(phase + TWO_PI * frequencies * DT + DT * coup, TWO_PI)
    return new_phase, new_amp
